# baseline (speedup 1.0000x reference)
import functools

import jax
import jax.numpy as jnp
from jax import lax
from jax.experimental import pallas as pl
from jax.experimental.pallas import tpu as pltpu
from jax.experimental.pallas import tpu_sc as plsc

N_NODES = 10000
D = 128
NPAD = 10240
N_TILES = 16
ROWS_PER_TILE = NPAD // N_TILES
NW = 32
CHUNK = 128
NBUF = 2
RB = 1024
_f32 = jnp.float32

_mesh = plsc.VectorSubcoreMesh(core_axis_name="c", subcore_axis_name="s")


def _zero_2d(ref, rows, cols):
    z = jnp.zeros((16,), _f32)
    cpl = cols // 16

    def body(i, carry):
        ref[i // cpl, pl.ds((i % cpl) * 16, 16)] = z
        return carry

    lax.fori_loop(0, rows * cpl, body, 0)


def _zero_1d(ref, n):
    z = jnp.zeros((16,), _f32)

    def body(i, carry):
        ref[pl.ds(i * 16, 16)] = z
        return carry

    lax.fori_loop(0, n // 16, body, 0)


def _copy_out(acc, out0, out1, c, tid):
    row0 = tid * ROWS_PER_TILE

    @pl.when(c == 0)
    def _():
        pltpu.sync_copy(acc.at[pl.ds(row0, ROWS_PER_TILE)],
                        out0.at[pl.ds(row0, ROWS_PER_TILE)])

    @pl.when(c == 1)
    def _():
        pltpu.sync_copy(acc.at[pl.ds(row0, ROWS_PER_TILE)],
                        out1.at[pl.ds(row0, ROWS_PER_TILE)])


def _deg_body(dst_hbm, out0, out1, dst_v, ones_v, zbuf, acc):
    c = lax.axis_index("c")
    s = lax.axis_index("s")
    wid = s * 2 + c
    cpw = dst_hbm.shape[1]

    pltpu.sync_copy(dst_hbm.at[wid], dst_v)
    one = jnp.ones((16,), _f32)
    for k in range(CHUNK // 16):
        ones_v[pl.ds(k * 16, 16)] = one
    _zero_1d(zbuf, ROWS_PER_TILE)
    pltpu.sync_copy(zbuf, acc.at[pl.ds(s * ROWS_PER_TILE, ROWS_PER_TILE)])
    plsc.subcore_barrier()

    def body(j, carry):
        pltpu.sync_copy(ones_v, acc.at[dst_v.at[j]], add=True)
        return carry

    lax.fori_loop(0, cpw, body, 0)
    plsc.subcore_barrier()
    _copy_out(acc, out0, out1, c, s)


G = 16


def _agg_body(g_hbm, src_hbm, dst_hbm, out0, out1,
              src_i0, src_i1, dst_i0, dst_i1,
              buf0, buf1, acc,
              semi0, semi1, semr0, semr1,
              sems0, sems1):
    c = lax.axis_index("c")
    s = lax.axis_index("s")
    wid = s * 2 + c
    cpw = src_hbm.shape[1]
    n_sup = cpw // G
    src_sb = (src_i0, src_i1)
    dst_sb = (dst_i0, dst_i1)
    bufs = (buf0, buf1)
    semi = (semi0, semi1)
    semr = (semr0, semr1)
    sems = (sems0, sems1)

    def issue_idx(sup):
        p = sup % 2
        pltpu.async_copy(src_hbm.at[wid].at[pl.ds(sup * G, G)], src_sb[p], semi[p])
        pltpu.async_copy(dst_hbm.at[wid].at[pl.ds(sup * G, G)], dst_sb[p], semi[p])

    def wait_idx(sup):
        p = sup % 2
        pltpu.make_async_copy(src_hbm.at[wid].at[pl.ds(0, G)], src_sb[p], semi[p]).wait()
        pltpu.make_async_copy(dst_hbm.at[wid].at[pl.ds(0, G)], dst_sb[p], semi[p]).wait()

    def issue_gather(j):
        b = j % NBUF
        pltpu.async_copy(g_hbm.at[src_sb[(j // G) % 2].at[j % G]], bufs[b], semr[b])

    def drain(sem, b):
        pltpu.make_async_copy(g_hbm.at[pl.ds(0, CHUNK)], bufs[b], sem).wait()

    issue_idx(0)
    issue_idx(1)

    _zero_2d(buf0, CHUNK, D)
    nz = ROWS_PER_TILE // CHUNK
    for k in range(nz):
        pltpu.async_copy(buf0, acc.at[pl.ds(s * ROWS_PER_TILE + k * CHUNK, CHUNK)],
                         sems0)
    for k in range(nz):
        drain(sems0, 0)
    plsc.subcore_barrier()

    wait_idx(0)
    issue_gather(0)
    issue_gather(1)

    for k in range(cpw):
        b = k % NBUF
        sup = k // G
        drain(semr[b], b)
        pltpu.async_copy(bufs[b], acc.at[dst_sb[sup % 2].at[k % G]],
                         sems[b], add=True)
        if k + 2 < cpw:
            s2 = (k + 2) // G
            if (k + 2) % G == 0:
                wait_idx(s2)
            if k + 2 - NBUF >= 0:
                drain(sems[(k + 2) % NBUF], (k + 2) % NBUF)
            issue_gather(k + 2)
        if k % G == 1 and sup >= 1 and (sup + 1) < n_sup:
            issue_idx(sup + 1)

    for k in range(cpw - NBUF, cpw):
        drain(sems[k % NBUF], k % NBUF)
    plsc.subcore_barrier()
    _copy_out(acc, out0, out1, c, s)


def _deg_call(dst_p):
    cpw = dst_p.shape[1]
    k = pl.kernel(
        _deg_body,
        out_type=[jax.ShapeDtypeStruct((NPAD,), _f32),
                  jax.ShapeDtypeStruct((NPAD,), _f32)],
        mesh=_mesh,
        scratch_types=[
            pltpu.VMEM((cpw, CHUNK), jnp.int32),
            pltpu.VMEM((CHUNK,), _f32),
            pltpu.VMEM((ROWS_PER_TILE,), _f32),
            pltpu.VMEM_SHARED((NPAD,), _f32),
        ],
    )
    return k(dst_p)


def _agg_call(g, src_p, dst_p):
    cpw = src_p.shape[1]
    k = pl.kernel(
        _agg_body,
        out_type=[jax.ShapeDtypeStruct((NPAD, D), _f32),
                  jax.ShapeDtypeStruct((NPAD, D), _f32)],
        mesh=_mesh,
        scratch_types=(
            [pltpu.VMEM((G, CHUNK), jnp.int32)] * 4
            + [pltpu.VMEM((CHUNK, D), _f32)] * NBUF
            + [pltpu.VMEM_SHARED((NPAD, D), _f32)]
            + [pltpu.SemaphoreType.DMA] * (2 + 2 * NBUF)
        ),
    )
    return k(g, src_p, dst_p)


def _dinv_block(d0_ref, d1_ref, i):
    deg = 1.0 + d0_ref[pl.ds(i * RB, RB)] + d1_ref[pl.ds(i * RB, RB)]
    return lax.rsqrt(deg)[:, None]


def _tc1_body(x_ref, w_ref, d0_ref, d1_ref, g_ref):
    i = pl.program_id(0)
    dinv = _dinv_block(d0_ref, d1_ref, i)
    g_ref[...] = jnp.dot(x_ref[...], w_ref[...],
                         preferred_element_type=_f32) * dinv


def _tc2_body(p0_ref, p1_ref, g1_ref, d0_ref, d1_ref, b1_ref, w2_ref, g2_ref):
    i = pl.program_id(0)
    dinv = _dinv_block(d0_ref, d1_ref, i)
    h = dinv * (p0_ref[...] + p1_ref[...] + g1_ref[...]) + b1_ref[...][None, :]
    a = jnp.maximum(h, 0.0)
    g2_ref[...] = jnp.dot(a, w2_ref[...], preferred_element_type=_f32) * dinv


def _tc3_body(p0_ref, p1_ref, g2_ref, d0_ref, d1_ref, b2_ref, o_ref):
    i = pl.program_id(0)
    dinv = _dinv_block(d0_ref, d1_ref, i)
    o_ref[...] = dinv * (p0_ref[...] + p1_ref[...] + g2_ref[...]) + b2_ref[...][None, :]


_row_spec = pl.BlockSpec((RB, D), lambda i: (i, 0))
_full_vec = pl.BlockSpec((NPAD,), lambda i: (0,))
_full_mat = pl.BlockSpec((D, D), lambda i: (0, 0))
_full_b = pl.BlockSpec((D,), lambda i: (0,))
_grid = (NPAD // RB,)


def _tc1(xp, W1, d0, d1):
    return pl.pallas_call(
        _tc1_body,
        grid=_grid,
        in_specs=[_row_spec, _full_mat, _full_vec, _full_vec],
        out_specs=_row_spec,
        out_shape=jax.ShapeDtypeStruct((NPAD, D), _f32),
    )(xp, W1, d0, d1)


def _tc2(p0, p1, g1, d0, d1, b1, W2):
    return pl.pallas_call(
        _tc2_body,
        grid=_grid,
        in_specs=[_row_spec, _row_spec, _row_spec, _full_vec, _full_vec,
                  _full_b, _full_mat],
        out_specs=_row_spec,
        out_shape=jax.ShapeDtypeStruct((NPAD, D), _f32),
    )(p0, p1, g1, d0, d1, b1, W2)


def _tc3(p0, p1, g2, d0, d1, b2):
    return pl.pallas_call(
        _tc3_body,
        grid=_grid,
        in_specs=[_row_spec, _row_spec, _row_spec, _full_vec, _full_vec,
                  _full_b],
        out_specs=_row_spec,
        out_shape=jax.ShapeDtypeStruct((NPAD, D), _f32),
    )(p0, p1, g2, d0, d1, b2)


def kernel(x, edge_index, W1, b1, W2, b2):
    src = edge_index[0].astype(jnp.int32)
    dst = edge_index[1].astype(jnp.int32)
    e = src.shape[0]
    cpw = -(-e // (NW * CHUNK))
    cpw = -(-cpw // G) * G
    epad = NW * cpw * CHUNK
    ar = jnp.arange(epad - e, dtype=jnp.int32)
    src_p = jnp.concatenate([src, ar % N_NODES]).reshape(NW, cpw, CHUNK)
    dst_p = jnp.concatenate(
        [dst, N_NODES + ar % (NPAD - N_NODES)]).reshape(NW, cpw, CHUNK)
    d0, d1 = _deg_call(dst_p)
    g1 = _tc1(x, W1, d0, d1)
    p0, p1 = _agg_call(g1, src_p, dst_p)
    g2 = _tc2(p0, p1, g1, d0, d1, b1, W2)
    q0, q1 = _agg_call(g2, src_p, dst_p)
    out = _tc3(q0, q1, g2, d0, d1, b2)
    return out[:N_NODES]


# speedup vs baseline: 1.0155x; 1.0155x over previous
import functools

import jax
import jax.numpy as jnp
from jax import lax
from jax.experimental import pallas as pl
from jax.experimental.pallas import tpu as pltpu
from jax.experimental.pallas import tpu_sc as plsc

N_NODES = 10000
D = 128
NPAD = 10240
N_TILES = 16
ROWS_PER_TILE = NPAD // N_TILES
NW = 32
CHUNK = 128
NBUF = 2
RB = 1024
_f32 = jnp.float32

_mesh = plsc.VectorSubcoreMesh(core_axis_name="c", subcore_axis_name="s")


def _zero_2d(ref, rows, cols):
    z = jnp.zeros((16,), _f32)
    cpl = cols // 16

    def body(i, carry):
        ref[i // cpl, pl.ds((i % cpl) * 16, 16)] = z
        return carry

    lax.fori_loop(0, rows * cpl, body, 0)


def _zero_1d(ref, n):
    z = jnp.zeros((16,), _f32)

    def body(i, carry):
        ref[pl.ds(i * 16, 16)] = z
        return carry

    lax.fori_loop(0, n // 16, body, 0)


def _copy_out(acc, out0, out1, c, tid):
    row0 = tid * ROWS_PER_TILE

    @pl.when(c == 0)
    def _():
        pltpu.sync_copy(acc.at[pl.ds(row0, ROWS_PER_TILE)],
                        out0.at[pl.ds(row0, ROWS_PER_TILE)])

    @pl.when(c == 1)
    def _():
        pltpu.sync_copy(acc.at[pl.ds(row0, ROWS_PER_TILE)],
                        out1.at[pl.ds(row0, ROWS_PER_TILE)])


def _deg_body(dst_hbm, out0, out1, dst_v, ones_v, zbuf, acc, sem):
    c = lax.axis_index("c")
    s = lax.axis_index("s")
    wid = s * 2 + c
    cpw = dst_hbm.shape[1]

    pltpu.sync_copy(dst_hbm.at[wid], dst_v)
    one = jnp.ones((16,), _f32)
    for k in range(CHUNK // 16):
        ones_v[pl.ds(k * 16, 16)] = one
    _zero_1d(zbuf, ROWS_PER_TILE)
    pltpu.sync_copy(zbuf, acc.at[pl.ds(s * ROWS_PER_TILE, ROWS_PER_TILE)])
    plsc.subcore_barrier()

    K = 20
    for base in range(0, cpw, K):
        for t in range(K):
            pltpu.async_copy(ones_v, acc.at[dst_v.at[base + t]], sem, add=True)
        for t in range(K):
            pltpu.make_async_copy(ones_v, acc.at[pl.ds(0, CHUNK)], sem).wait()

    plsc.subcore_barrier()
    _copy_out(acc, out0, out1, c, s)


G = 16


def _agg_body(g_hbm, src_hbm, dst_hbm, out0, out1,
              src_i0, src_i1, dst_i0, dst_i1,
              buf0, buf1, acc,
              semi0, semi1, semr0, semr1,
              sems0, sems1):
    c = lax.axis_index("c")
    s = lax.axis_index("s")
    wid = s * 2 + c
    cpw = src_hbm.shape[1]
    n_sup = cpw // G
    src_sb = (src_i0, src_i1)
    dst_sb = (dst_i0, dst_i1)
    bufs = (buf0, buf1)
    semi = (semi0, semi1)
    semr = (semr0, semr1)
    sems = (sems0, sems1)

    def issue_idx(sup):
        p = sup % 2
        pltpu.async_copy(src_hbm.at[wid].at[pl.ds(sup * G, G)], src_sb[p], semi[p])
        pltpu.async_copy(dst_hbm.at[wid].at[pl.ds(sup * G, G)], dst_sb[p], semi[p])

    def wait_idx(sup):
        p = sup % 2
        pltpu.make_async_copy(src_hbm.at[wid].at[pl.ds(0, G)], src_sb[p], semi[p]).wait()
        pltpu.make_async_copy(dst_hbm.at[wid].at[pl.ds(0, G)], dst_sb[p], semi[p]).wait()

    def issue_gather(j):
        b = j % NBUF
        pltpu.async_copy(g_hbm.at[src_sb[(j // G) % 2].at[j % G]], bufs[b], semr[b])

    def drain(sem, b):
        pltpu.make_async_copy(g_hbm.at[pl.ds(0, CHUNK)], bufs[b], sem).wait()

    issue_idx(0)
    issue_idx(1)

    _zero_2d(buf0, CHUNK, D)
    nz = ROWS_PER_TILE // CHUNK
    for k in range(nz):
        pltpu.async_copy(buf0, acc.at[pl.ds(s * ROWS_PER_TILE + k * CHUNK, CHUNK)],
                         sems0)
    for k in range(nz):
        drain(sems0, 0)
    plsc.subcore_barrier()

    wait_idx(0)
    issue_gather(0)
    issue_gather(1)

    for k in range(cpw):
        b = k % NBUF
        sup = k // G
        drain(semr[b], b)
        pltpu.async_copy(bufs[b], acc.at[dst_sb[sup % 2].at[k % G]],
                         sems[b], add=True)
        if k + 2 < cpw:
            s2 = (k + 2) // G
            if (k + 2) % G == 0:
                wait_idx(s2)
            if k + 2 - NBUF >= 0:
                drain(sems[(k + 2) % NBUF], (k + 2) % NBUF)
            issue_gather(k + 2)
        if k % G == 1 and sup >= 1 and (sup + 1) < n_sup:
            issue_idx(sup + 1)

    for k in range(cpw - NBUF, cpw):
        drain(sems[k % NBUF], k % NBUF)
    plsc.subcore_barrier()
    _copy_out(acc, out0, out1, c, s)


def _deg_call(dst_p):
    cpw = dst_p.shape[1]
    k = pl.kernel(
        _deg_body,
        out_type=[jax.ShapeDtypeStruct((NPAD,), _f32),
                  jax.ShapeDtypeStruct((NPAD,), _f32)],
        mesh=_mesh,
        scratch_types=[
            pltpu.VMEM((cpw, CHUNK), jnp.int32),
            pltpu.VMEM((CHUNK,), _f32),
            pltpu.VMEM((ROWS_PER_TILE,), _f32),
            pltpu.VMEM_SHARED((NPAD,), _f32),
            pltpu.SemaphoreType.DMA,
        ],
    )
    return k(dst_p)


def _agg_call(g, src_p, dst_p):
    cpw = src_p.shape[1]
    k = pl.kernel(
        _agg_body,
        out_type=[jax.ShapeDtypeStruct((NPAD, D), _f32),
                  jax.ShapeDtypeStruct((NPAD, D), _f32)],
        mesh=_mesh,
        scratch_types=(
            [pltpu.VMEM((G, CHUNK), jnp.int32)] * 4
            + [pltpu.VMEM((CHUNK, D), _f32)] * NBUF
            + [pltpu.VMEM_SHARED((NPAD, D), _f32)]
            + [pltpu.SemaphoreType.DMA] * (2 + 2 * NBUF)
        ),
    )
    return k(g, src_p, dst_p)


def _dinv_block(d0_ref, d1_ref, i):
    deg = 1.0 + d0_ref[pl.ds(i * RB, RB)] + d1_ref[pl.ds(i * RB, RB)]
    return lax.rsqrt(deg)[:, None]


def _tc1_body(x_ref, w_ref, d0_ref, d1_ref, g_ref):
    i = pl.program_id(0)
    dinv = _dinv_block(d0_ref, d1_ref, i)
    g_ref[...] = jnp.dot(x_ref[...], w_ref[...],
                         preferred_element_type=_f32) * dinv


def _tc2_body(p0_ref, p1_ref, g1_ref, d0_ref, d1_ref, b1_ref, w2_ref, g2_ref):
    i = pl.program_id(0)
    dinv = _dinv_block(d0_ref, d1_ref, i)
    h = dinv * (p0_ref[...] + p1_ref[...] + g1_ref[...]) + b1_ref[...][None, :]
    a = jnp.maximum(h, 0.0)
    g2_ref[...] = jnp.dot(a, w2_ref[...], preferred_element_type=_f32) * dinv


def _tc3_body(p0_ref, p1_ref, g2_ref, d0_ref, d1_ref, b2_ref, o_ref):
    i = pl.program_id(0)
    dinv = _dinv_block(d0_ref, d1_ref, i)
    o_ref[...] = dinv * (p0_ref[...] + p1_ref[...] + g2_ref[...]) + b2_ref[...][None, :]


_row_spec = pl.BlockSpec((RB, D), lambda i: (i, 0))
_full_vec = pl.BlockSpec((NPAD,), lambda i: (0,))
_full_mat = pl.BlockSpec((D, D), lambda i: (0, 0))
_full_b = pl.BlockSpec((D,), lambda i: (0,))
_grid = (NPAD // RB,)


def _tc1(xp, W1, d0, d1):
    return pl.pallas_call(
        _tc1_body,
        grid=_grid,
        in_specs=[_row_spec, _full_mat, _full_vec, _full_vec],
        out_specs=_row_spec,
        out_shape=jax.ShapeDtypeStruct((NPAD, D), _f32),
    )(xp, W1, d0, d1)


def _tc2(p0, p1, g1, d0, d1, b1, W2):
    return pl.pallas_call(
        _tc2_body,
        grid=_grid,
        in_specs=[_row_spec, _row_spec, _row_spec, _full_vec, _full_vec,
                  _full_b, _full_mat],
        out_specs=_row_spec,
        out_shape=jax.ShapeDtypeStruct((NPAD, D), _f32),
    )(p0, p1, g1, d0, d1, b1, W2)


def _tc3(p0, p1, g2, d0, d1, b2):
    return pl.pallas_call(
        _tc3_body,
        grid=_grid,
        in_specs=[_row_spec, _row_spec, _row_spec, _full_vec, _full_vec,
                  _full_b],
        out_specs=_row_spec,
        out_shape=jax.ShapeDtypeStruct((NPAD, D), _f32),
    )(p0, p1, g2, d0, d1, b2)


def kernel(x, edge_index, W1, b1, W2, b2):
    src = edge_index[0].astype(jnp.int32)
    dst = edge_index[1].astype(jnp.int32)
    e = src.shape[0]
    cpw = -(-e // (NW * CHUNK))
    cpw = -(-cpw // G) * G
    epad = NW * cpw * CHUNK
    ar = jnp.arange(epad - e, dtype=jnp.int32)
    src_p = jnp.concatenate([src, ar % N_NODES]).reshape(NW, cpw, CHUNK)
    dst_p = jnp.concatenate(
        [dst, N_NODES + ar % (NPAD - N_NODES)]).reshape(NW, cpw, CHUNK)
    d0, d1 = _deg_call(dst_p)
    g1 = _tc1(x, W1, d0, d1)
    p0, p1 = _agg_call(g1, src_p, dst_p)
    g2 = _tc2(p0, p1, g1, d0, d1, b1, W2)
    q0, q1 = _agg_call(g2, src_p, dst_p)
    out = _tc3(q0, q1, g2, d0, d1, b2)
    return out[:N_NODES]


# speedup vs baseline: 1.0281x; 1.0124x over previous
import functools

import jax
import jax.numpy as jnp
from jax import lax
from jax.experimental import pallas as pl
from jax.experimental.pallas import tpu as pltpu
from jax.experimental.pallas import tpu_sc as plsc

N_NODES = 10000
D = 128
NPAD = 10240
N_TILES = 16
ROWS_PER_TILE = NPAD // N_TILES
NW = 32
CHUNK = 128
NBUF = 2
RB = 1024
_f32 = jnp.float32

_mesh = plsc.VectorSubcoreMesh(core_axis_name="c", subcore_axis_name="s")


def _zero_2d(ref, rows, cols):
    z = jnp.zeros((16,), _f32)
    cpl = cols // 16

    def body(i, carry):
        ref[i // cpl, pl.ds((i % cpl) * 16, 16)] = z
        return carry

    lax.fori_loop(0, rows * cpl, body, 0)


def _zero_1d(ref, n):
    z = jnp.zeros((16,), _f32)

    def body(i, carry):
        ref[pl.ds(i * 16, 16)] = z
        return carry

    lax.fori_loop(0, n // 16, body, 0)


def _copy_out(acc, out0, out1, c, tid):
    row0 = tid * ROWS_PER_TILE

    @pl.when(c == 0)
    def _():
        pltpu.sync_copy(acc.at[pl.ds(row0, ROWS_PER_TILE)],
                        out0.at[pl.ds(row0, ROWS_PER_TILE)])

    @pl.when(c == 1)
    def _():
        pltpu.sync_copy(acc.at[pl.ds(row0, ROWS_PER_TILE)],
                        out1.at[pl.ds(row0, ROWS_PER_TILE)])


def _deg_body(dst_hbm, out0, out1, dst_v, ones_v, zbuf, acc, sem):
    c = lax.axis_index("c")
    s = lax.axis_index("s")
    wid = s * 2 + c
    cpw = dst_hbm.shape[1]

    pltpu.sync_copy(dst_hbm.at[wid], dst_v)
    one = jnp.ones((16,), _f32)
    for k in range(CHUNK // 16):
        ones_v[pl.ds(k * 16, 16)] = one
    _zero_1d(zbuf, ROWS_PER_TILE)
    pltpu.sync_copy(zbuf, acc.at[pl.ds(s * ROWS_PER_TILE, ROWS_PER_TILE)])
    plsc.subcore_barrier()

    K = 20
    for base in range(0, cpw, K):
        for t in range(K):
            pltpu.async_copy(ones_v, acc.at[dst_v.at[base + t]], sem, add=True)
        for t in range(K):
            pltpu.make_async_copy(ones_v, acc.at[pl.ds(0, CHUNK)], sem).wait()

    plsc.subcore_barrier()
    _copy_out(acc, out0, out1, c, s)


G = 16


def _agg_body(g_hbm, src_hbm, dst_hbm, out0, out1,
              src_i0, src_i1, dst_i0, dst_i1,
              buf0, buf1, acc,
              semi0, semi1, semr0, semr1,
              sems0, sems1):
    c = lax.axis_index("c")
    s = lax.axis_index("s")
    wid = s * 2 + c
    cpw = src_hbm.shape[1]
    n_sup = cpw // G
    src_sb = (src_i0, src_i1)
    dst_sb = (dst_i0, dst_i1)
    bufs = (buf0, buf1)
    semi = (semi0, semi1)
    semr = (semr0, semr1)
    sems = (sems0, sems1)

    def issue_idx(sup):
        p = sup % 2
        pltpu.async_copy(src_hbm.at[wid].at[pl.ds(sup * G, G)], src_sb[p], semi[p])
        pltpu.async_copy(dst_hbm.at[wid].at[pl.ds(sup * G, G)], dst_sb[p], semi[p])

    def wait_idx(sup):
        p = sup % 2
        pltpu.make_async_copy(src_hbm.at[wid].at[pl.ds(0, G)], src_sb[p], semi[p]).wait()
        pltpu.make_async_copy(dst_hbm.at[wid].at[pl.ds(0, G)], dst_sb[p], semi[p]).wait()

    def issue_gather(j):
        b = j % NBUF
        pltpu.async_copy(g_hbm.at[src_sb[(j // G) % 2].at[j % G]], bufs[b], semr[b])

    def drain(sem, b):
        pltpu.make_async_copy(g_hbm.at[pl.ds(0, CHUNK)], bufs[b], sem).wait()

    issue_idx(0)
    issue_idx(1)

    _zero_2d(buf0, CHUNK, D)
    nz = ROWS_PER_TILE // CHUNK
    for k in range(nz):
        pltpu.async_copy(buf0, acc.at[pl.ds(s * ROWS_PER_TILE + k * CHUNK, CHUNK)],
                         sems0)
    for k in range(nz):
        drain(sems0, 0)
    plsc.subcore_barrier()

    wait_idx(0)
    issue_gather(0)
    issue_gather(1)

    for k in range(cpw):
        b = k % NBUF
        sup = k // G
        drain(semr[b], b)
        pltpu.async_copy(bufs[b], acc.at[dst_sb[sup % 2].at[k % G]],
                         sems[b], add=True)
        if k + 2 < cpw:
            s2 = (k + 2) // G
            if (k + 2) % G == 0:
                wait_idx(s2)
            if k + 2 - NBUF >= 0:
                drain(sems[(k + 2) % NBUF], (k + 2) % NBUF)
            issue_gather(k + 2)
        if k % G == 1 and sup >= 1 and (sup + 1) < n_sup:
            issue_idx(sup + 1)

    for k in range(cpw - NBUF, cpw):
        drain(sems[k % NBUF], k % NBUF)
    plsc.subcore_barrier()
    _copy_out(acc, out0, out1, c, s)


def _deg_call(dst_p):
    cpw = dst_p.shape[1]
    k = pl.kernel(
        _deg_body,
        out_type=[jax.ShapeDtypeStruct((NPAD,), _f32),
                  jax.ShapeDtypeStruct((NPAD,), _f32)],
        mesh=_mesh,
        scratch_types=[
            pltpu.VMEM((cpw, CHUNK), jnp.int32),
            pltpu.VMEM((CHUNK,), _f32),
            pltpu.VMEM((ROWS_PER_TILE,), _f32),
            pltpu.VMEM_SHARED((NPAD,), _f32),
            pltpu.SemaphoreType.DMA,
        ],
    )
    return k(dst_p)


def _agg_call(g, src_p, dst_p):
    cpw = src_p.shape[1]
    k = pl.kernel(
        _agg_body,
        out_type=[jax.ShapeDtypeStruct((NPAD, D), _f32),
                  jax.ShapeDtypeStruct((NPAD, D), _f32)],
        mesh=_mesh,
        scratch_types=(
            [pltpu.VMEM((G, CHUNK), jnp.int32)] * 4
            + [pltpu.VMEM((CHUNK, D), _f32)] * NBUF
            + [pltpu.VMEM_SHARED((NPAD, D), _f32)]
            + [pltpu.SemaphoreType.DMA] * (2 + 2 * NBUF)
        ),
    )
    return k(g, src_p, dst_p)


def _tc1_body(x_ref, w_ref, d0_ref, d1_ref, g_ref, dinv_ref):
    i = pl.program_id(0)
    deg = 1.0 + d0_ref[pl.ds(i * RB, RB)] + d1_ref[pl.ds(i * RB, RB)]
    dinv = lax.rsqrt(deg)[:, None]
    dinv_ref[...] = dinv
    g_ref[...] = jnp.dot(x_ref[...], w_ref[...],
                         preferred_element_type=_f32) * dinv


def _tc2_body(p0_ref, p1_ref, g1_ref, dinv_ref, b1_ref, w2_ref, g2_ref):
    dinv = dinv_ref[...]
    h = dinv * (p0_ref[...] + p1_ref[...] + g1_ref[...]) + b1_ref[...][None, :]
    a = jnp.maximum(h, 0.0)
    g2_ref[...] = jnp.dot(a, w2_ref[...], preferred_element_type=_f32) * dinv


def _tc3_body(p0_ref, p1_ref, g2_ref, dinv_ref, b2_ref, o_ref):
    o_ref[...] = dinv_ref[...] * (p0_ref[...] + p1_ref[...] + g2_ref[...]) \
        + b2_ref[...][None, :]


_row_spec = pl.BlockSpec((RB, D), lambda i: (i, 0))
_dinv_spec = pl.BlockSpec((RB, 1), lambda i: (i, 0))
_full_vec = pl.BlockSpec((NPAD,), lambda i: (0,))
_full_mat = pl.BlockSpec((D, D), lambda i: (0, 0))
_full_b = pl.BlockSpec((D,), lambda i: (0,))
_grid = (NPAD // RB,)
RB3 = 2000
_row3_spec = pl.BlockSpec((RB3, D), lambda i: (i, 0))
_dinv3_spec = pl.BlockSpec((RB3, 1), lambda i: (i, 0))
_grid3 = (N_NODES // RB3,)


def _tc1(x, W1, d0, d1):
    return pl.pallas_call(
        _tc1_body,
        grid=_grid,
        in_specs=[_row_spec, _full_mat, _full_vec, _full_vec],
        out_specs=[_row_spec, _dinv_spec],
        out_shape=[jax.ShapeDtypeStruct((NPAD, D), _f32),
                   jax.ShapeDtypeStruct((NPAD, 1), _f32)],
    )(x, W1, d0, d1)


def _tc2(p0, p1, g1, dinv, b1, W2):
    return pl.pallas_call(
        _tc2_body,
        grid=_grid,
        in_specs=[_row_spec, _row_spec, _row_spec, _dinv_spec,
                  _full_b, _full_mat],
        out_specs=_row_spec,
        out_shape=jax.ShapeDtypeStruct((NPAD, D), _f32),
    )(p0, p1, g1, dinv, b1, W2)


def _tc3(p0, p1, g2, dinv, b2):
    return pl.pallas_call(
        _tc3_body,
        grid=_grid3,
        in_specs=[_row3_spec, _row3_spec, _row3_spec, _dinv3_spec, _full_b],
        out_specs=_row3_spec,
        out_shape=jax.ShapeDtypeStruct((N_NODES, D), _f32),
    )(p0, p1, g2, dinv, b2)


def kernel(x, edge_index, W1, b1, W2, b2):
    src = edge_index[0].astype(jnp.int32)
    dst = edge_index[1].astype(jnp.int32)
    e = src.shape[0]
    cpw = -(-e // (NW * CHUNK))
    cpw = -(-cpw // G) * G
    epad = NW * cpw * CHUNK
    ar = jnp.arange(epad - e, dtype=jnp.int32)
    src_p = jnp.concatenate([src, ar % N_NODES]).reshape(NW, cpw, CHUNK)
    dst_p = jnp.concatenate(
        [dst, N_NODES + ar % (NPAD - N_NODES)]).reshape(NW, cpw, CHUNK)
    d0, d1 = _deg_call(dst_p)
    g1, dinv = _tc1(x, W1, d0, d1)
    p0, p1 = _agg_call(g1, src_p, dst_p)
    g2 = _tc2(p0, p1, g1, dinv, b1, W2)
    q0, q1 = _agg_call(g2, src_p, dst_p)
    return _tc3(q0, q1, g2, dinv, b2)


# speedup vs baseline: 1.0447x; 1.0162x over previous
import functools

import jax
import jax.numpy as jnp
from jax import lax
from jax.experimental import pallas as pl
from jax.experimental.pallas import tpu as pltpu
from jax.experimental.pallas import tpu_sc as plsc

N_NODES = 10000
D = 128
NPAD = 10240
N_TILES = 16
ROWS_PER_TILE = NPAD // N_TILES
NW = 32
CHUNK = 128
NBUF = 2
RB = 2048
_f32 = jnp.float32

_mesh = plsc.VectorSubcoreMesh(core_axis_name="c", subcore_axis_name="s")


def _zero_2d(ref, rows, cols):
    z = jnp.zeros((16,), _f32)
    cpl = cols // 16

    def body(i, carry):
        ref[i // cpl, pl.ds((i % cpl) * 16, 16)] = z
        return carry

    lax.fori_loop(0, rows * cpl, body, 0)


def _zero_1d(ref, n):
    z = jnp.zeros((16,), _f32)

    def body(i, carry):
        ref[pl.ds(i * 16, 16)] = z
        return carry

    lax.fori_loop(0, n // 16, body, 0)


def _copy_out(acc, out0, out1, c, tid):
    row0 = tid * ROWS_PER_TILE

    @pl.when(c == 0)
    def _():
        pltpu.sync_copy(acc.at[pl.ds(row0, ROWS_PER_TILE)],
                        out0.at[pl.ds(row0, ROWS_PER_TILE)])

    @pl.when(c == 1)
    def _():
        pltpu.sync_copy(acc.at[pl.ds(row0, ROWS_PER_TILE)],
                        out1.at[pl.ds(row0, ROWS_PER_TILE)])


def _deg_body(dst_hbm, out0, out1, dst_v, ones_v, zbuf, acc, sem):
    c = lax.axis_index("c")
    s = lax.axis_index("s")
    wid = s * 2 + c
    cpw = dst_hbm.shape[1]

    pltpu.sync_copy(dst_hbm.at[wid], dst_v)
    one = jnp.ones((16,), _f32)
    for k in range(CHUNK // 16):
        ones_v[pl.ds(k * 16, 16)] = one
    _zero_1d(zbuf, ROWS_PER_TILE)
    pltpu.sync_copy(zbuf, acc.at[pl.ds(s * ROWS_PER_TILE, ROWS_PER_TILE)])
    plsc.subcore_barrier()

    K = 40
    for base in range(0, cpw, K):
        for t in range(K):
            pltpu.async_copy(ones_v, acc.at[dst_v.at[base + t]], sem, add=True)
        for t in range(K):
            pltpu.make_async_copy(ones_v, acc.at[pl.ds(0, CHUNK)], sem).wait()

    plsc.subcore_barrier()
    _copy_out(acc, out0, out1, c, s)


G = 16


def _agg_body(g_hbm, src_hbm, dst_hbm, out0, out1,
              src_i0, src_i1, dst_i0, dst_i1,
              buf0, buf1, acc,
              semi0, semi1, semr0, semr1,
              sems0, sems1):
    c = lax.axis_index("c")
    s = lax.axis_index("s")
    wid = s * 2 + c
    cpw = src_hbm.shape[1]
    n_sup = cpw // G
    src_sb = (src_i0, src_i1)
    dst_sb = (dst_i0, dst_i1)
    bufs = (buf0, buf1)
    semi = (semi0, semi1)
    semr = (semr0, semr1)
    sems = (sems0, sems1)

    def issue_idx(sup):
        p = sup % 2
        pltpu.async_copy(src_hbm.at[wid].at[pl.ds(sup * G, G)], src_sb[p], semi[p])
        pltpu.async_copy(dst_hbm.at[wid].at[pl.ds(sup * G, G)], dst_sb[p], semi[p])

    def wait_idx(sup):
        p = sup % 2
        pltpu.make_async_copy(src_hbm.at[wid].at[pl.ds(0, G)], src_sb[p], semi[p]).wait()
        pltpu.make_async_copy(dst_hbm.at[wid].at[pl.ds(0, G)], dst_sb[p], semi[p]).wait()

    def issue_gather(j):
        b = j % NBUF
        pltpu.async_copy(g_hbm.at[src_sb[(j // G) % 2].at[j % G]], bufs[b], semr[b])

    def drain(sem, b):
        pltpu.make_async_copy(g_hbm.at[pl.ds(0, CHUNK)], bufs[b], sem).wait()

    issue_idx(0)
    issue_idx(1)

    _zero_2d(buf0, CHUNK, D)
    nz = ROWS_PER_TILE // CHUNK
    for k in range(nz):
        pltpu.async_copy(buf0, acc.at[pl.ds(s * ROWS_PER_TILE + k * CHUNK, CHUNK)],
                         sems0)
    for k in range(nz):
        drain(sems0, 0)
    plsc.subcore_barrier()

    wait_idx(0)
    issue_gather(0)
    issue_gather(1)

    for k in range(cpw):
        b = k % NBUF
        sup = k // G
        drain(semr[b], b)
        pltpu.async_copy(bufs[b], acc.at[dst_sb[sup % 2].at[k % G]],
                         sems[b], add=True)
        if k + 2 < cpw:
            s2 = (k + 2) // G
            if (k + 2) % G == 0:
                wait_idx(s2)
            if k + 2 - NBUF >= 0:
                drain(sems[(k + 2) % NBUF], (k + 2) % NBUF)
            issue_gather(k + 2)
        if k % G == 1 and sup >= 1 and (sup + 1) < n_sup:
            issue_idx(sup + 1)

    for k in range(cpw - NBUF, cpw):
        drain(sems[k % NBUF], k % NBUF)
    plsc.subcore_barrier()
    _copy_out(acc, out0, out1, c, s)


def _deg_call(dst_p):
    cpw = dst_p.shape[1]
    k = pl.kernel(
        _deg_body,
        out_type=[jax.ShapeDtypeStruct((NPAD,), _f32),
                  jax.ShapeDtypeStruct((NPAD,), _f32)],
        mesh=_mesh,
        scratch_types=[
            pltpu.VMEM((cpw, CHUNK), jnp.int32),
            pltpu.VMEM((CHUNK,), _f32),
            pltpu.VMEM((ROWS_PER_TILE,), _f32),
            pltpu.VMEM_SHARED((NPAD,), _f32),
            pltpu.SemaphoreType.DMA,
        ],
    )
    return k(dst_p)


def _agg_call(g, src_p, dst_p):
    cpw = src_p.shape[1]
    k = pl.kernel(
        _agg_body,
        out_type=[jax.ShapeDtypeStruct((NPAD, D), _f32),
                  jax.ShapeDtypeStruct((NPAD, D), _f32)],
        mesh=_mesh,
        scratch_types=(
            [pltpu.VMEM((G, CHUNK), jnp.int32)] * 4
            + [pltpu.VMEM((CHUNK, D), _f32)] * NBUF
            + [pltpu.VMEM_SHARED((NPAD, D), _f32)]
            + [pltpu.SemaphoreType.DMA] * (2 + 2 * NBUF)
        ),
    )
    return k(g, src_p, dst_p)


def _tc1_body(x_ref, w_ref, d0_ref, d1_ref, g_ref, dinv_ref):
    i = pl.program_id(0)
    deg = 1.0 + d0_ref[pl.ds(i * RB, RB)] + d1_ref[pl.ds(i * RB, RB)]
    dinv = lax.rsqrt(deg)[:, None]
    dinv_ref[...] = dinv
    g_ref[...] = jnp.dot(x_ref[...], w_ref[...],
                         preferred_element_type=_f32) * dinv


def _tc2_body(p0_ref, p1_ref, g1_ref, dinv_ref, b1_ref, w2_ref, g2_ref):
    dinv = dinv_ref[...]
    h = dinv * (p0_ref[...] + p1_ref[...] + g1_ref[...]) + b1_ref[...][None, :]
    a = jnp.maximum(h, 0.0)
    g2_ref[...] = jnp.dot(a, w2_ref[...], preferred_element_type=_f32) * dinv


def _tc3_body(p0_ref, p1_ref, g2_ref, dinv_ref, b2_ref, o_ref):
    o_ref[...] = dinv_ref[...] * (p0_ref[...] + p1_ref[...] + g2_ref[...]) \
        + b2_ref[...][None, :]


_row_spec = pl.BlockSpec((RB, D), lambda i: (i, 0))
_dinv_spec = pl.BlockSpec((RB, 1), lambda i: (i, 0))
_full_vec = pl.BlockSpec((NPAD,), lambda i: (0,))
_full_mat = pl.BlockSpec((D, D), lambda i: (0, 0))
_full_b = pl.BlockSpec((D,), lambda i: (0,))
_grid = (NPAD // RB,)
RB3 = 2000
_row3_spec = pl.BlockSpec((RB3, D), lambda i: (i, 0))
_dinv3_spec = pl.BlockSpec((RB3, 1), lambda i: (i, 0))
_grid3 = (N_NODES // RB3,)


def _tc1(x, W1, d0, d1):
    return pl.pallas_call(
        _tc1_body,
        grid=_grid,
        in_specs=[_row_spec, _full_mat, _full_vec, _full_vec],
        out_specs=[_row_spec, _dinv_spec],
        out_shape=[jax.ShapeDtypeStruct((NPAD, D), _f32),
                   jax.ShapeDtypeStruct((NPAD, 1), _f32)],
    )(x, W1, d0, d1)


def _tc2(p0, p1, g1, dinv, b1, W2):
    return pl.pallas_call(
        _tc2_body,
        grid=_grid,
        in_specs=[_row_spec, _row_spec, _row_spec, _dinv_spec,
                  _full_b, _full_mat],
        out_specs=_row_spec,
        out_shape=jax.ShapeDtypeStruct((NPAD, D), _f32),
    )(p0, p1, g1, dinv, b1, W2)


def _tc3(p0, p1, g2, dinv, b2):
    return pl.pallas_call(
        _tc3_body,
        grid=_grid3,
        in_specs=[_row3_spec, _row3_spec, _row3_spec, _dinv3_spec, _full_b],
        out_specs=_row3_spec,
        out_shape=jax.ShapeDtypeStruct((N_NODES, D), _f32),
    )(p0, p1, g2, dinv, b2)


def kernel(x, edge_index, W1, b1, W2, b2):
    src = edge_index[0].astype(jnp.int32)
    dst = edge_index[1].astype(jnp.int32)
    e = src.shape[0]
    cpw = -(-e // (NW * CHUNK))
    cpw = -(-cpw // G) * G
    epad = NW * cpw * CHUNK
    ar = jnp.arange(epad - e, dtype=jnp.int32)
    src_p = jnp.concatenate([src, ar % N_NODES]).reshape(NW, cpw, CHUNK)
    dst_p = jnp.concatenate(
        [dst, N_NODES + ar % (NPAD - N_NODES)]).reshape(NW, cpw, CHUNK)
    d0, d1 = _deg_call(dst_p)
    g1, dinv = _tc1(x, W1, d0, d1)
    p0, p1 = _agg_call(g1, src_p, dst_p)
    g2 = _tc2(p0, p1, g1, dinv, b1, W2)
    q0, q1 = _agg_call(g2, src_p, dst_p)
    return _tc3(q0, q1, g2, dinv, b2)
